# baseline (speedup 1.0000x reference)
import jax
import jax.numpy as jnp
from jax import lax
from jax.experimental import pallas as pl
from jax.experimental.pallas import tpu as pltpu
from jax.experimental.pallas import tpu_sc as plsc

N_USERS = 1000000
N_ITEMS = 1000000
EMB = 64
BATCH = 16384

NC = 2
NS = 16
L = 16
NW = NC * NS

W = 128
CPW = 66
COLS_PER_W = W * CPW
SC_COLS = COLS_PER_W * NW
SC_Q = SC_COLS // 128

BLK = 24576
TC_SKIP = SC_COLS // BLK
TC_GRID = 30
TC_Q = TC_GRID * BLK // 128

B_PER_W = BATCH // NW
CHUNK = 128
NCHUNK = B_PER_W // CHUNK


def _rsqrt_sc(x):
    i = plsc.bitcast(x, jnp.int32)
    i = jnp.int32(0x5F3759DF) - lax.shift_right_logical(i, 1)
    y = plsc.bitcast(i, jnp.float32)
    for _ in range(3):
        y = y * (1.5 - 0.5 * x * y * y)
    return y


def _sc_dense(ut_hbm, uet_hbm, it_hbm, iet_hbm, vu_hbm, vi_hbm,
              au, bu, ai, bi, au2, bu2, ai2, bi2, vu_loc, vi_loc,
              sem_a, sem_b):
    wid = lax.axis_index("s") * NC + lax.axis_index("c")
    cbase = wid * COLS_PER_W
    zero = jnp.zeros((L,), jnp.float32)

    def start(chunk, bufs, sem):
        c0 = cbase + jnp.minimum(chunk, CPW - 1) * W
        sl = pl.ds(c0, W)
        pltpu.async_copy(ut_hbm.at[:, sl], bufs[0], sem)
        pltpu.async_copy(uet_hbm.at[:, sl], bufs[1], sem)
        pltpu.async_copy(it_hbm.at[:, sl], bufs[2], sem)
        pltpu.async_copy(iet_hbm.at[:, sl], bufs[3], sem)

    def drain(bufs, sem):
        for b in bufs:
            pltpu.make_async_copy(ut_hbm.at[:, pl.ds(0, W)], b, sem).wait()

    def chunk_pair(abuf, bbuf, vloc, t):
        def grp(g, carry):
            sl = pl.ds(g * L, L)
            dot = [zero] * 4
            na = [zero] * 4
            nb = [zero] * 4
            for c in range(EMB):
                k = c & 3
                va = abuf[c, sl]
                vb = bbuf[c, sl]
                dot[k] = dot[k] + va * vb
                na[k] = na[k] + va * va
                nb[k] = nb[k] + vb * vb
            d = (dot[0] + dot[1]) + (dot[2] + dot[3])
            a2 = (na[0] + na[1]) + (na[2] + na[3])
            b2 = (nb[0] + nb[1]) + (nb[2] + nb[3])
            cos = d * _rsqrt_sc(jnp.maximum(a2 * b2, jnp.float32(1e-16)))
            vloc[pl.ds(t * W + g * L, L)] = (cos + 1.0) * 0.5
            return carry

        lax.fori_loop(0, W // L, grp, 0)

    bufs_a = (au, bu, ai, bi)
    bufs_b = (au2, bu2, ai2, bi2)

    start(0, bufs_a, sem_a)

    def body(t2, carry):
        ca = t2 * 2
        start(ca + 1, bufs_b, sem_b)
        drain(bufs_a, sem_a)
        chunk_pair(au, bu, vu_loc, ca)
        chunk_pair(ai, bi, vi_loc, ca)
        start(ca + 2, bufs_a, sem_a)
        drain(bufs_b, sem_b)
        chunk_pair(au2, bu2, vu_loc, ca + 1)
        chunk_pair(ai2, bi2, vi_loc, ca + 1)
        return carry

    lax.fori_loop(0, CPW // 2, body, 0)
    drain(bufs_a, sem_a)

    out_sl = pl.ds(wid * COLS_PER_W, COLS_PER_W)
    pltpu.sync_copy(vu_loc, vu_hbm.at[out_sl])
    pltpu.sync_copy(vi_loc, vi_hbm.at[out_sl])


def _tc_dense(ut_ref, uet_ref, it_ref, iet_ref, vu_ref, vi_ref):
    def pair(a_ref, b_ref):
        a = a_ref[...]
        b = b_ref[...]
        dot = jnp.sum(a * b, axis=0)
        na = jnp.sum(a * a, axis=0)
        nb = jnp.sum(b * b, axis=0)
        cos = dot * lax.rsqrt(jnp.maximum(na * nb, jnp.float32(1e-16)))
        return (cos + 1.0) * 0.5

    vu_ref[...] = pair(ut_ref, uet_ref).reshape(BLK // 128, 128)
    vi_ref[...] = pair(it_ref, iet_ref).reshape(BLK // 128, 128)


def _sc_gather(vut_hbm, vit_hbm, vus_hbm, vis_hbm, u_hbm, i_hbm, out_hbm,
               um_v, im_v, qt_v, qs_v, rt, rs, out_v, sem):
    wid = lax.axis_index("s") * NC + lax.axis_index("c")
    base = wid * B_PER_W

    for j in range(NCHUNK):
        pltpu.sync_copy(u_hbm.at[pl.ds(base + j * CHUNK, CHUNK)], um_v.at[j])
        pltpu.sync_copy(i_hbm.at[pl.ds(base + j * CHUNK, CHUNK)], im_v.at[j])

    for idx_v, v_tc_hbm, v_sc_hbm, first in (
        (um_v, vut_hbm, vus_hbm, True),
        (im_v, vit_hbm, vis_hbm, False),
    ):
        for j in range(NCHUNK):
            for o in range(CHUNK // L):
                sl = pl.ds(o * L, L)
                q = lax.shift_right_logical(idx_v[j, sl], 7)
                qt_v[j, sl] = jnp.maximum(q - SC_Q, 0)
                qs_v[j, sl] = jnp.minimum(q, SC_Q - 1)
        for j in range(NCHUNK):
            ct = pltpu.async_copy(v_tc_hbm.at[qt_v.at[j]], rt, sem)
            cs = pltpu.async_copy(v_sc_hbm.at[qs_v.at[j]], rs, sem)
            ct.wait()
            cs.wait()
            for g in range(CHUNK // L):
                sl = pl.ds(g * L, L)
                rows = lax.iota(jnp.int32, L) + g * L
                idx = idx_v[j, sl]
                m = jnp.bitwise_and(idx, 127)
                vt = plsc.load_gather(rt, [rows, m])
                vs = plsc.load_gather(rs, [rows, m])
                val = jnp.where(idx < SC_COLS, vs, vt)
                osl = pl.ds(j * CHUNK + g * L, L)
                if first:
                    out_v[osl] = val
                else:
                    out_v[osl] = out_v[osl] + val

    pltpu.sync_copy(out_v, out_hbm.at[pl.ds(base, B_PER_W)])


@jax.jit
def kernel(user_emb, item_emb, u_emb_ema, i_emb_ema, u, i):
    u = u.astype(jnp.int32)
    i = i.astype(jnp.int32)
    ut, it_, uet, iet = user_emb.T, item_emb.T, u_emb_ema.T, i_emb_ema.T
    mesh = plsc.VectorSubcoreMesh(core_axis_name="c", subcore_axis_name="s")
    f32 = jnp.float32

    sc_dense = pl.kernel(
        _sc_dense,
        out_type=[jax.ShapeDtypeStruct((SC_COLS,), f32)] * 2,
        mesh=mesh,
        compiler_params=pltpu.CompilerParams(needs_layout_passes=False),
        scratch_types=(
            [pltpu.VMEM((EMB, W), f32)] * 8
            + [pltpu.VMEM((COLS_PER_W,), f32)] * 2
            + [pltpu.SemaphoreType.DMA, pltpu.SemaphoreType.DMA]
        ),
    )
    vu_sc, vi_sc = sc_dense(ut, uet, it_, iet)

    tc_dense = pl.pallas_call(
        _tc_dense,
        grid=(TC_GRID,),
        in_specs=[pl.BlockSpec((EMB, BLK), lambda g: (0, g + TC_SKIP))] * 4,
        out_specs=[pl.BlockSpec((BLK // 128, 128), lambda g: (g, 0))] * 2,
        out_shape=[jax.ShapeDtypeStruct((TC_Q, 128), f32)] * 2,
    )
    vu_tc, vi_tc = tc_dense(ut, uet, it_, iet)

    sc_gather = pl.kernel(
        _sc_gather,
        out_type=jax.ShapeDtypeStruct((BATCH,), f32),
        mesh=mesh,
        compiler_params=pltpu.CompilerParams(needs_layout_passes=False),
        scratch_types=[
            pltpu.VMEM((NCHUNK, CHUNK), jnp.int32),
            pltpu.VMEM((NCHUNK, CHUNK), jnp.int32),
            pltpu.VMEM((NCHUNK, CHUNK), jnp.int32),
            pltpu.VMEM((NCHUNK, CHUNK), jnp.int32),
            pltpu.VMEM((CHUNK, 128), f32),
            pltpu.VMEM((CHUNK, 128), f32),
            pltpu.VMEM((B_PER_W,), f32),
            pltpu.SemaphoreType.DMA,
        ],
    )
    return sc_gather(vu_tc, vi_tc,
                     vu_sc.reshape(SC_Q, 128), vi_sc.reshape(SC_Q, 128),
                     u, i)


# speedup vs baseline: 3.5641x; 3.5641x over previous
import jax
import jax.numpy as jnp
from jax import lax
from jax.experimental import pallas as pl
from jax.experimental.pallas import tpu as pltpu
from jax.experimental.pallas import tpu_sc as plsc

N_USERS = 1000000
N_ITEMS = 1000000
EMB = 64
BATCH = 16384

BLK = 24576
GRID = 41
N_PAD = GRID * BLK
QROWS = N_PAD // 128

NC = 2
NS = 16
L = 16
NW = NC * NS
B_PER_W = BATCH // NW
CHUNK = 128
NCHUNK = B_PER_W // CHUNK


def _tc_dense(ut_ref, uet_ref, it_ref, iet_ref, vu_ref, vi_ref):
    def pair(a_ref, b_ref):
        a = a_ref[...]
        b = b_ref[...]
        dot = jnp.sum(a * b, axis=0)
        na = jnp.sum(a * a, axis=0)
        nb = jnp.sum(b * b, axis=0)
        cos = dot * lax.rsqrt(jnp.maximum(na * nb, jnp.float32(1e-16)))
        return (cos + 1.0) * 0.5

    vu_ref[...] = pair(ut_ref, uet_ref).reshape(BLK // 128, 128)
    vi_ref[...] = pair(it_ref, iet_ref).reshape(BLK // 128, 128)


def _sc_gather(vu_hbm, vi_hbm, u_hbm, i_hbm, out_hbm,
               um_v, im_v, uq_v, iq_v, ru, ri, out_v, sem):
    wid = lax.axis_index("s") * NC + lax.axis_index("c")
    base = wid * B_PER_W

    for j in range(NCHUNK):
        pltpu.sync_copy(u_hbm.at[pl.ds(base + j * CHUNK, CHUNK)], um_v.at[j])
        pltpu.sync_copy(i_hbm.at[pl.ds(base + j * CHUNK, CHUNK)], im_v.at[j])
    for j in range(NCHUNK):
        for o in range(CHUNK // L):
            sl = pl.ds(o * L, L)
            uq_v[j, sl] = lax.shift_right_logical(um_v[j, sl], 7)
            iq_v[j, sl] = lax.shift_right_logical(im_v[j, sl], 7)

    for j in range(NCHUNK):
        cu = pltpu.async_copy(vu_hbm.at[uq_v.at[j]], ru, sem)
        ci = pltpu.async_copy(vi_hbm.at[iq_v.at[j]], ri, sem)
        cu.wait()
        ci.wait()
        for g in range(CHUNK // L):
            sl = pl.ds(g * L, L)
            rows = lax.iota(jnp.int32, L) + g * L
            mu = jnp.bitwise_and(um_v[j, sl], 127)
            mi = jnp.bitwise_and(im_v[j, sl], 127)
            vmu = plsc.load_gather(ru, [rows, mu])
            vmi = plsc.load_gather(ri, [rows, mi])
            out_v[pl.ds(j * CHUNK + g * L, L)] = vmu + vmi

    pltpu.sync_copy(out_v, out_hbm.at[pl.ds(base, B_PER_W)])


@jax.jit
def kernel(user_emb, item_emb, u_emb_ema, i_emb_ema, u, i):
    u = u.astype(jnp.int32)
    i = i.astype(jnp.int32)

    tc = pl.pallas_call(
        _tc_dense,
        grid=(GRID,),
        in_specs=[pl.BlockSpec((EMB, BLK), lambda g: (0, g))] * 4,
        out_specs=[pl.BlockSpec((BLK // 128, 128), lambda g: (g, 0))] * 2,
        out_shape=[jax.ShapeDtypeStruct((QROWS, 128), jnp.float32)] * 2,
    )
    vu2, vi2 = tc(user_emb.T, u_emb_ema.T, item_emb.T, i_emb_ema.T)

    mesh = plsc.VectorSubcoreMesh(core_axis_name="c", subcore_axis_name="s")
    sc = pl.kernel(
        _sc_gather,
        out_type=jax.ShapeDtypeStruct((BATCH,), jnp.float32),
        mesh=mesh,
        compiler_params=pltpu.CompilerParams(needs_layout_passes=False),
        scratch_types=[
            pltpu.VMEM((NCHUNK, CHUNK), jnp.int32),
            pltpu.VMEM((NCHUNK, CHUNK), jnp.int32),
            pltpu.VMEM((NCHUNK, CHUNK), jnp.int32),
            pltpu.VMEM((NCHUNK, CHUNK), jnp.int32),
            pltpu.VMEM((CHUNK, 128), jnp.float32),
            pltpu.VMEM((CHUNK, 128), jnp.float32),
            pltpu.VMEM((B_PER_W,), jnp.float32),
            pltpu.SemaphoreType.DMA,
        ],
    )
    return sc(vu2, vi2, u, i)
